# baseline (speedup 1.0000x reference)
import functools

import jax
import jax.numpy as jnp
from jax import lax
from jax.experimental import pallas as pl
from jax.experimental.pallas import tpu as pltpu, tpu_sc as plsc

N_NODES = 100000
D_FEAT = 128
BATCH = 65536

NC = 2
NS = 16
NW = NC * NS
CHUNK = 128
ROWS_PER_W = BATCH // NW
N_CHUNKS = ROWS_PER_W // CHUNK
SUP = 2
N_SUP = N_CHUNKS // SUP


def _make_gather():
    mesh = plsc.VectorSubcoreMesh(core_axis_name="c", subcore_axis_name="s")

    K = 2
    NSLOT = 3

    @functools.partial(
        pl.kernel,
        mesh=mesh,
        out_type=jax.ShapeDtypeStruct((NW, N_SUP, 1, SUP * CHUNK, D_FEAT),
                                      jnp.float32),
        scratch_types=[
            pltpu.VMEM((N_SUP, 1, SUP * CHUNK), jnp.int32),
        ] + [pltpu.VMEM((1, SUP * CHUNK, D_FEAT), jnp.float32)] * K
          + [pltpu.VMEM_SHARED((NS, NSLOT, CHUNK, D_FEAT), jnp.float32)]
          + [pltpu.SemaphoreType.DMA] * (K + 2 * NSLOT),
    )
    def gather(table_hbm, idx_hbm, out_hbm, idx_v, *rest):
        bufs = rest[:K]
        spbuf = rest[K]
        gsems = rest[K + 1:2 * K + 1]
        xsems = rest[2 * K + 1:2 * K + 1 + NSLOT]
        dsems = rest[2 * K + 1 + NSLOT:2 * K + 1 + 2 * NSLOT]
        cid = lax.axis_index("c")
        sid = lax.axis_index("s")
        wid = sid * NC + cid

        pltpu.sync_copy(idx_hbm.at[wid], idx_v)

        def fire_gather(s):
            return pltpu.async_copy(
                table_hbm.at[idx_v.at[s]], bufs[s % K], gsems[s % K])

        def fire_stage(c):
            s, h, r = c // SUP, c % SUP, c % NSLOT
            return pltpu.async_copy(
                bufs[s % K].at[0, pl.ds(h * CHUNK, CHUNK)],
                spbuf.at[sid, r], xsems[r])

        def fire_write(c):
            s, h, r = c // SUP, c % SUP, c % NSLOT
            return pltpu.async_copy(
                spbuf.at[sid, r],
                out_hbm.at[wid, s, 0, pl.ds(h * CHUNK, CHUNK)], dsems[r])

        NCH = N_SUP * SUP
        gcp = [None] * K
        xcp = {}
        dcp = {}
        gcp[0] = fire_gather(0)
        for s in range(N_SUP):
            for h in range(SUP):
                c = SUP * (s - 1) + h
                if c >= 0:
                    xcp.pop(c).wait()
                    dcp[c] = fire_write(c)
            if s + 1 < N_SUP:
                gcp[(s + 1) % K] = fire_gather(s + 1)
            gcp[s % K].wait()
            for h in range(SUP):
                c = SUP * s + h
                if c - NSLOT in dcp:
                    dcp.pop(c - NSLOT).wait()
                xcp[c] = fire_stage(c)
        for c in (NCH - 2, NCH - 1):
            xcp.pop(c).wait()
            dcp[c] = fire_write(c)
        for c in sorted(dcp):
            dcp[c].wait()

    return gather


_gather = _make_gather()


def kernel(node_features, source_nodes, timestamps, n_layers):
    del timestamps, n_layers
    idx = source_nodes.reshape(NW, N_SUP, 1, SUP * CHUNK)
    table = node_features.reshape(1, N_NODES, D_FEAT)
    return _gather(table, idx).reshape(BATCH, D_FEAT)


# speedup vs baseline: 1.0714x; 1.0714x over previous
import functools

import jax
import jax.numpy as jnp
from jax import lax
from jax.experimental import pallas as pl
from jax.experimental.pallas import tpu as pltpu, tpu_sc as plsc

N_NODES = 100000
D_FEAT = 128
BATCH = 65536

NC = 2
NS = 16
NW = NC * NS
CHUNK = 128
ROWS_PER_W = BATCH // NW
N_CHUNKS = ROWS_PER_W // CHUNK
SUP = 2
N_SUP = N_CHUNKS // SUP


def _make_gather():
    mesh = plsc.VectorSubcoreMesh(core_axis_name="c", subcore_axis_name="s")

    K = 3
    LEAD = 2

    @functools.partial(
        pl.kernel,
        mesh=mesh,
        out_type=jax.ShapeDtypeStruct((NW, N_SUP, 1, SUP * CHUNK, D_FEAT),
                                      jnp.float32),
        scratch_types=[
            pltpu.VMEM((N_SUP, 1, SUP * CHUNK), jnp.int32),
        ] + [pltpu.VMEM((1, SUP * CHUNK, D_FEAT), jnp.float32)] * K
          + [pltpu.SemaphoreType.DMA] * (2 * K),
    )
    def gather(table_hbm, idx_hbm, out_hbm, idx_v, *bufs_and_sems):
        bufs = bufs_and_sems[:K]
        gsems = bufs_and_sems[K:2 * K]
        osems = bufs_and_sems[2 * K:3 * K]
        wid = lax.axis_index("s") * NC + lax.axis_index("c")

        pltpu.sync_copy(idx_hbm.at[wid], idx_v)

        def fire_gather(s):
            b = s % K
            return pltpu.async_copy(
                table_hbm.at[idx_v.at[s]], bufs[b],
                gsems[b])

        gcp = [None] * K
        ocp = [None] * K
        for m in range(LEAD):
            gcp[m % K] = fire_gather(m)
        for s in range(N_SUP):
            m = s + LEAD
            if m < N_SUP:
                b = m % K
                if ocp[b] is not None:
                    ocp[b].wait()
                    ocp[b] = None
                gcp[b] = fire_gather(m)
            gcp[s % K].wait()
            ocp[s % K] = pltpu.async_copy(
                bufs[s % K], out_hbm.at[wid, s], osems[s % K])
        for b in range(K):
            if ocp[b] is not None:
                ocp[b].wait()

    return gather


_gather = _make_gather()


def kernel(node_features, source_nodes, timestamps, n_layers):
    del timestamps, n_layers
    idx = source_nodes.reshape(NW, N_SUP, 1, SUP * CHUNK)
    table = node_features.reshape(1, N_NODES, D_FEAT)
    return _gather(table, idx).reshape(BATCH, D_FEAT)
